# baseline (speedup 1.0000x reference)
import functools

import jax
import jax.numpy as jnp
from jax import lax
from jax.experimental import pallas as pl
from jax.experimental.pallas import tpu as pltpu
from jax.experimental.pallas import tpu_sc as plsc

N = 10000
E = 160000
H = 256

NC = 2
NS = 16
NW = NC * NS
L = 16

f32 = jnp.float32
i32 = jnp.int32


EPW = E // NW
GC = 128
NFULL = EPW // GC
TAIL = EPW - NFULL * GC

_mesh = plsc.VectorSubcoreMesh(
    core_axis_name="c", subcore_axis_name="s", num_cores=NC, num_subcores=NS)
_sc_params = pltpu.CompilerParams(needs_layout_passes=False)


@functools.partial(
    pl.kernel,
    out_type=[jax.ShapeDtypeStruct((E, H), f32),
              jax.ShapeDtypeStruct((E, H), f32)],
    mesh=_mesh,
    scratch_types=[
        pltpu.VMEM((EPW,), i32),
        pltpu.VMEM((EPW,), i32),
        pltpu.VMEM((GC, H), f32),
        pltpu.VMEM((GC, H), f32),
        pltpu.SemaphoreType.DMA,
        pltpu.SemaphoreType.DMA,
    ],
    compiler_params=_sc_params,
)
def _gather2(ps_hbm, pr_hbm, sidx_hbm, ridx_hbm, gs_hbm, gr_hbm,
             sidx_v, ridx_v, rs_v, rr_v, sem_s, sem_r):
    wid = lax.axis_index("s") * NC + lax.axis_index("c")
    base = wid * EPW
    pltpu.sync_copy(sidx_hbm.at[pl.ds(base, EPW)], sidx_v)
    pltpu.sync_copy(ridx_hbm.at[pl.ds(base, EPW)], ridx_v)

    @pl.loop(0, NFULL)
    def _chunk(i):
        off = i * GC
        cs = pltpu.async_copy(ps_hbm.at[sidx_v.at[pl.ds(off, GC)]], rs_v, sem_s)
        cr = pltpu.async_copy(pr_hbm.at[ridx_v.at[pl.ds(off, GC)]], rr_v, sem_r)
        cs.wait()
        cr.wait()
        pltpu.sync_copy(rs_v, gs_hbm.at[pl.ds(base + off, GC)])
        pltpu.sync_copy(rr_v, gr_hbm.at[pl.ds(base + off, GC)])

    toff = NFULL * GC
    cs = pltpu.async_copy(
        ps_hbm.at[sidx_v.at[pl.ds(toff, TAIL)]], rs_v.at[pl.ds(0, TAIL)], sem_s)
    cr = pltpu.async_copy(
        pr_hbm.at[ridx_v.at[pl.ds(toff, TAIL)]], rr_v.at[pl.ds(0, TAIL)], sem_r)
    cs.wait()
    cr.wait()
    pltpu.sync_copy(rs_v.at[pl.ds(0, TAIL)], gs_hbm.at[pl.ds(base + toff, TAIL)])
    pltpu.sync_copy(rr_v.at[pl.ds(0, TAIL)], gr_hbm.at[pl.ds(base + toff, TAIL)])


NPW = 320
NPAD = NW * NPW
RC = 2000
NCHUNK = E // RC
RB = 64
IDXSZ = RC + RB


@functools.partial(
    pl.kernel,
    out_type=jax.ShapeDtypeStruct((NPAD, H), f32),
    mesh=_mesh,
    scratch_types=[
        pltpu.VMEM((NPW + 8, H), f32),
        pltpu.VMEM((RC,), i32),
        pltpu.VMEM((RC,), i32),
        pltpu.VMEM((IDXSZ + L,), i32),
        pltpu.VMEM((IDXSZ + L,), i32),
        pltpu.VMEM((RB, H), f32),
        pltpu.VMEM((RB, H), f32),
        pltpu.SemaphoreType.DMA,
        pltpu.SemaphoreType.DMA,
        pltpu.SemaphoreType.DMA,
    ],
    compiler_params=_sc_params,
)
def _segsum(msg_hbm, ridx_hbm, agg_hbm, acc_v, rix_a, rix_b, eid_v, loc_v,
            rows_a, rows_b, sem_r, sem_a, sem_b):
    wid = lax.axis_index("s") * NC + lax.axis_index("c")
    lo = wid * NPW
    hi = jnp.minimum(lo + NPW, N)
    lanes = lax.iota(i32, L)
    zf = jnp.zeros((L,), f32)
    zi = jnp.zeros((L,), i32)

    @pl.loop(0, NPW + 8)
    def _zacc(j):
        for c in range(H // L):
            acc_v[j, pl.ds(c * L, L)] = zf

    @pl.loop(0, (IDXSZ + L) // L, unroll=8)
    def _zeid(j):
        eid_v[pl.ds(j * L, L)] = zi

    pltpu.async_copy(ridx_hbm.at[pl.ds(0, RC)], rix_a, sem_r).wait()

    @pl.loop(0, NCHUNK)
    def _chunk(ci):
        ebase = ci * RC
        even = (ci % 2) == 0

        @pl.when(ci + 1 < NCHUNK)
        def _pref():
            @pl.when(even)
            def _pb():
                pltpu.async_copy(
                    ridx_hbm.at[pl.ds(ebase + RC, RC)], rix_b, sem_r)
            @pl.when(jnp.logical_not(even))
            def _pa():
                pltpu.async_copy(
                    ridx_hbm.at[pl.ds(ebase + RC, RC)], rix_a, sem_r)

        def _with_buf(rix_v):
            def _cbody(j, cnt_vec):
                r = rix_v[pl.ds(j * L, L)]
                m = (r >= lo) & (r < hi)
                mi = m.astype(i32)
                pos = plsc.cumsum(mi) - 1
                tgt = jnp.where(m, cnt_vec + pos, IDXSZ)
                plsc.store_scatter(eid_v, [tgt], ebase + j * L + lanes)
                plsc.store_scatter(loc_v, [tgt], r - lo)
                return cnt_vec + plsc.all_reduce_population_count(m)

            cnt_vec = lax.fori_loop(0, RC // L, _cbody, jnp.zeros((L,), i32),
                                    unroll=4)
            cnt = jnp.max(cnt_vec)
            nblk = (cnt + (RB - 1)) // RB

            @pl.when(nblk > 0)
            def _go():
                pltpu.async_copy(
                    msg_hbm.at[eid_v.at[pl.ds(0, RB)]], rows_a, sem_a)

                def _gbody(b, _):
                    beven = (b % 2) == 0

                    @pl.when(b + 1 < nblk)
                    def _fire():
                        boff = (b + 1) * RB
                        @pl.when(beven)
                        def _fb():
                            pltpu.async_copy(
                                msg_hbm.at[eid_v.at[pl.ds(boff, RB)]],
                                rows_b, sem_b)
                        @pl.when(jnp.logical_not(beven))
                        def _fa():
                            pltpu.async_copy(
                                msg_hbm.at[eid_v.at[pl.ds(boff, RB)]],
                                rows_a, sem_a)

                    def _accum(rows_v, sem):
                        pltpu.make_async_copy(
                            msg_hbm.at[eid_v.at[pl.ds(0, RB)]], rows_v,
                            sem).wait()

                        def _abody(k, _):
                            base = b * RB + k * L
                            rvalid = (base + lanes) < cnt_vec
                            q = jnp.where(rvalid, loc_v[pl.ds(base, L)], NPW)
                            rowi = k * L + lanes

                            @pl.loop(0, H, unroll=16)
                            def _col(c):
                                cv = jnp.full((L,), c, i32)
                                vals = plsc.load_gather(rows_v, [rowi, cv])
                                plsc.addupdate_scatter(acc_v, [q, cv], vals)
                            return 0

                        nk = jnp.minimum(cnt - b * RB, RB)
                        lax.fori_loop(0, (nk + (L - 1)) // L, _abody, 0)

                    @pl.when(beven)
                    def _da():
                        _accum(rows_a, sem_a)

                    @pl.when(jnp.logical_not(beven))
                    def _db():
                        _accum(rows_b, sem_b)

                    return 0

                lax.fori_loop(0, nblk, _gbody, 0)

        @pl.when(even)
        def _ea():
            _with_buf(rix_a)

        @pl.when(jnp.logical_not(even))
        def _eb():
            _with_buf(rix_b)

        @pl.when(ci + 1 < NCHUNK)
        def _wait_pref():
            @pl.when(even)
            def _wb():
                pltpu.make_async_copy(
                    ridx_hbm.at[pl.ds(0, RC)], rix_b, sem_r).wait()
            @pl.when(jnp.logical_not(even))
            def _wa():
                pltpu.make_async_copy(
                    ridx_hbm.at[pl.ds(0, RC)], rix_a, sem_r).wait()

    pltpu.sync_copy(acc_v.at[pl.ds(0, NPW)], agg_hbm.at[pl.ds(lo, NPW)])


NB_A = 400
NB_E = 400
EB = 1600


def _preproj_body(nf, ws, wr, ps, pr):
    x = nf[...]
    ps[...] = jnp.dot(x, ws[...], preferred_element_type=f32)
    pr[...] = jnp.dot(x, wr[...], preferred_element_type=f32)


def _ln(x, g, b):
    mu = jnp.mean(x, -1, keepdims=True)
    xc = x - mu
    var = jnp.mean(xc * xc, -1, keepdims=True)
    return xc * lax.rsqrt(var + 1e-5) * g + b


def _edge_body(gs, gr, ef, w0, w1, w2, b0, b1, b2, g, bt, msg, ne):
    efv = ef[...]
    x = gs[...] + gr[...] + jnp.dot(efv, w0[...], preferred_element_type=f32)
    x = jnp.maximum(x + b0[...], 0.0)
    x = jnp.maximum(jnp.dot(x, w1[...], preferred_element_type=f32) + b1[...], 0.0)
    x = jnp.dot(x, w2[...], preferred_element_type=f32) + b2[...]
    y = _ln(x, g[...], bt[...])
    msg[...] = y
    ne[...] = y + efv


def _node_body(nf, agg, w0a, w0b, w1, w2, b0, b1, b2, g, bt, out):
    nfv = nf[...]
    x = (jnp.dot(nfv, w0a[...], preferred_element_type=f32)
         + jnp.dot(agg[...], w0b[...], preferred_element_type=f32))
    x = jnp.maximum(x + b0[...], 0.0)
    x = jnp.maximum(jnp.dot(x, w1[...], preferred_element_type=f32) + b1[...], 0.0)
    x = jnp.dot(x, w2[...], preferred_element_type=f32) + b2[...]
    out[...] = _ln(x, g[...], bt[...]) + nfv


def _full(shape):
    return pl.BlockSpec(shape, lambda i: (0, 0))


def kernel(senders, receivers, node_features, edge_features,
           eW0, eb0, eW1, eb1, eW2, eb2, eg, ebt,
           nW0, nb0, nW1, nb1, nW2, nb2, ng, nbt):
    w_es, w_er, w_ee = eW0[0:H], eW0[H:2 * H], eW0[2 * H:3 * H]
    nW0a, nW0b = nW0[0:H], nW0[H:2 * H]
    eb0r, eb1r, eb2r = eb0.reshape(1, H), eb1.reshape(1, H), eb2.reshape(1, H)
    egr, ebtr = eg.reshape(1, H), ebt.reshape(1, H)
    nb0r, nb1r, nb2r = nb0.reshape(1, H), nb1.reshape(1, H), nb2.reshape(1, H)
    ngr, nbtr = ng.reshape(1, H), nbt.reshape(1, H)

    ps, pr = pl.pallas_call(
        _preproj_body,
        grid=(N // NB_A,),
        in_specs=[pl.BlockSpec((NB_A, H), lambda i: (i, 0)),
                  _full((H, H)), _full((H, H))],
        out_specs=[pl.BlockSpec((NB_A, H), lambda i: (i, 0))] * 2,
        out_shape=[jax.ShapeDtypeStruct((N, H), f32)] * 2,
    )(node_features, w_es, w_er)

    gs, gr = _gather2(ps, pr, senders, receivers)

    msg, new_edge = pl.pallas_call(
        _edge_body,
        grid=(E // EB,),
        in_specs=[pl.BlockSpec((EB, H), lambda i: (i, 0))] * 3
        + [_full((H, H))] * 3 + [_full((1, H))] * 5,
        out_specs=[pl.BlockSpec((EB, H), lambda i: (i, 0))] * 2,
        out_shape=[jax.ShapeDtypeStruct((E, H), f32)] * 2,
    )(gs, gr, edge_features, w_ee, eW1, eW2, eb0r, eb1r, eb2r, egr, ebtr)

    aggp = _segsum(msg, receivers)

    new_node = pl.pallas_call(
        _node_body,
        grid=(N // NB_E,),
        in_specs=[pl.BlockSpec((NB_E, H), lambda i: (i, 0))] * 2
        + [_full((H, H))] * 4 + [_full((1, H))] * 5,
        out_specs=pl.BlockSpec((NB_E, H), lambda i: (i, 0)),
        out_shape=jax.ShapeDtypeStruct((N, H), f32),
    )(node_features, aggp, nW0a, nW0b, nW1, nW2, nb0r, nb1r, nb2r, ngr, nbtr)

    return (new_node, new_edge)


# speedup vs baseline: 1.0953x; 1.0953x over previous
import functools

import jax
import jax.numpy as jnp
from jax import lax
from jax.experimental import pallas as pl
from jax.experimental.pallas import tpu as pltpu
from jax.experimental.pallas import tpu_sc as plsc

N = 10000
E = 160000
H = 256

NC = 2
NS = 16
NW = NC * NS
L = 16

f32 = jnp.float32
i32 = jnp.int32


EPW = E // NW
GC = 64
NFULL = EPW // GC
TAIL = EPW - NFULL * GC

_mesh = plsc.VectorSubcoreMesh(
    core_axis_name="c", subcore_axis_name="s", num_cores=NC, num_subcores=NS)
_sc_params = pltpu.CompilerParams(needs_layout_passes=False)


@functools.partial(
    pl.kernel,
    out_type=[jax.ShapeDtypeStruct((E, H), f32),
              jax.ShapeDtypeStruct((E, H), f32)],
    mesh=_mesh,
    scratch_types=[
        pltpu.VMEM((EPW,), i32),
        pltpu.VMEM((EPW,), i32),
        pltpu.VMEM((GC, H), f32),
        pltpu.VMEM((GC, H), f32),
        pltpu.VMEM((GC, H), f32),
        pltpu.VMEM((GC, H), f32),
        pltpu.VMEM((GC, H), f32),
        pltpu.VMEM((GC, H), f32),
        pltpu.SemaphoreType.DMA,
        pltpu.SemaphoreType.DMA,
        pltpu.SemaphoreType.DMA,
        pltpu.SemaphoreType.DMA,
        pltpu.SemaphoreType.DMA,
        pltpu.SemaphoreType.DMA,
        pltpu.SemaphoreType.DMA,
        pltpu.SemaphoreType.DMA,
        pltpu.SemaphoreType.DMA,
        pltpu.SemaphoreType.DMA,
        pltpu.SemaphoreType.DMA,
        pltpu.SemaphoreType.DMA,
    ],
    compiler_params=_sc_params,
)
def _gather2(ps_hbm, pr_hbm, sidx_hbm, ridx_hbm, gs_hbm, gr_hbm,
             sidx_v, ridx_v,
             bs0, bs1, bs2, br0, br1, br2,
             gs0, gs1, gs2, gr0, gr1, gr2,
             ws0, ws1, ws2, wr0, wr1, wr2):
    wid = lax.axis_index("s") * NC + lax.axis_index("c")
    base = wid * EPW
    bs = (bs0, bs1, bs2)
    br = (br0, br1, br2)
    gsem = (gs0, gs1, gs2)
    rsem = (gr0, gr1, gr2)
    wssem = (ws0, ws1, ws2)
    wrsem = (wr0, wr1, wr2)

    pltpu.sync_copy(sidx_hbm.at[pl.ds(base, EPW)], sidx_v)
    pltpu.sync_copy(ridx_hbm.at[pl.ds(base, EPW)], ridx_v)

    def fire(i, b):
        off = i * GC
        pltpu.async_copy(ps_hbm.at[sidx_v.at[pl.ds(off, GC)]], bs[b], gsem[b])
        pltpu.async_copy(pr_hbm.at[ridx_v.at[pl.ds(off, GC)]], br[b], rsem[b])

    def wait_gather(b):
        pltpu.make_async_copy(
            ps_hbm.at[sidx_v.at[pl.ds(0, GC)]], bs[b], gsem[b]).wait()
        pltpu.make_async_copy(
            pr_hbm.at[ridx_v.at[pl.ds(0, GC)]], br[b], rsem[b]).wait()

    def fire_wb(i, b):
        off = i * GC
        pltpu.async_copy(bs[b], gs_hbm.at[pl.ds(base + off, GC)], wssem[b])
        pltpu.async_copy(br[b], gr_hbm.at[pl.ds(base + off, GC)], wrsem[b])

    def wait_wb(b):
        pltpu.make_async_copy(
            bs[b], gs_hbm.at[pl.ds(base, GC)], wssem[b]).wait()
        pltpu.make_async_copy(
            br[b], gr_hbm.at[pl.ds(base, GC)], wrsem[b]).wait()

    for i in (0, 1):
        fire(i, i)

    @pl.loop(0, NFULL)
    def _body(i):
        for b in range(3):
            @pl.when((i % 3) == b)
            def _(b=b):
                wait_gather(b)
                fire_wb(i, b)

        @pl.when(i + 2 < NFULL)
        def _fnext():
            for b in range(3):
                @pl.when(((i + 2) % 3) == b)
                def _(b=b):
                    @pl.when(i >= 1)
                    def _w():
                        wait_wb(b)
                    fire(i + 2, b)

    for b in range(3):
        wait_wb(b)

    toff = NFULL * GC
    cs = pltpu.async_copy(
        ps_hbm.at[sidx_v.at[pl.ds(toff, TAIL)]], bs0.at[pl.ds(0, TAIL)], gs0)
    cr = pltpu.async_copy(
        pr_hbm.at[ridx_v.at[pl.ds(toff, TAIL)]], br0.at[pl.ds(0, TAIL)], gr0)
    cs.wait()
    cr.wait()
    pltpu.sync_copy(bs0.at[pl.ds(0, TAIL)], gs_hbm.at[pl.ds(base + toff, TAIL)])
    pltpu.sync_copy(br0.at[pl.ds(0, TAIL)], gr_hbm.at[pl.ds(base + toff, TAIL)])


NPW = 320
NPAD = NW * NPW
RC = 2000
NCHUNK = E // RC
RB = 64
IDXSZ = RC + RB


@functools.partial(
    pl.kernel,
    out_type=jax.ShapeDtypeStruct((NPAD, H), f32),
    mesh=_mesh,
    scratch_types=[
        pltpu.VMEM((NPW + 8, H), f32),
        pltpu.VMEM((RC,), i32),
        pltpu.VMEM((RC,), i32),
        pltpu.VMEM((IDXSZ + L,), i32),
        pltpu.VMEM((IDXSZ + L,), i32),
        pltpu.VMEM((RB, H), f32),
        pltpu.VMEM((RB, H), f32),
        pltpu.SemaphoreType.DMA,
        pltpu.SemaphoreType.DMA,
        pltpu.SemaphoreType.DMA,
    ],
    compiler_params=_sc_params,
)
def _segsum(msg_hbm, ridx_hbm, agg_hbm, acc_v, rix_a, rix_b, eid_v, loc_v,
            rows_a, rows_b, sem_r, sem_a, sem_b):
    wid = lax.axis_index("s") * NC + lax.axis_index("c")
    lo = wid * NPW
    hi = jnp.minimum(lo + NPW, N)
    lanes = lax.iota(i32, L)
    zf = jnp.zeros((L,), f32)
    zi = jnp.zeros((L,), i32)

    @pl.loop(0, NPW + 8)
    def _zacc(j):
        for c in range(H // L):
            acc_v[j, pl.ds(c * L, L)] = zf

    @pl.loop(0, (IDXSZ + L) // L, unroll=8)
    def _zeid(j):
        eid_v[pl.ds(j * L, L)] = zi

    pltpu.async_copy(ridx_hbm.at[pl.ds(0, RC)], rix_a, sem_r).wait()

    @pl.loop(0, NCHUNK)
    def _chunk(ci):
        ebase = ci * RC
        even = (ci % 2) == 0

        @pl.when(ci + 1 < NCHUNK)
        def _pref():
            @pl.when(even)
            def _pb():
                pltpu.async_copy(
                    ridx_hbm.at[pl.ds(ebase + RC, RC)], rix_b, sem_r)
            @pl.when(jnp.logical_not(even))
            def _pa():
                pltpu.async_copy(
                    ridx_hbm.at[pl.ds(ebase + RC, RC)], rix_a, sem_r)

        def _with_buf(rix_v):
            def _cbody(j, cnt_vec):
                r = rix_v[pl.ds(j * L, L)]
                m = (r >= lo) & (r < hi)
                mi = m.astype(i32)
                pos = plsc.cumsum(mi) - 1
                tgt = jnp.where(m, cnt_vec + pos, IDXSZ)
                plsc.store_scatter(eid_v, [tgt], ebase + j * L + lanes)
                plsc.store_scatter(loc_v, [tgt], r - lo)
                return cnt_vec + plsc.all_reduce_population_count(m)

            cnt_vec = lax.fori_loop(0, RC // L, _cbody, jnp.zeros((L,), i32),
                                    unroll=4)
            cnt = jnp.max(cnt_vec)
            nblk = (cnt + (RB - 1)) // RB

            @pl.when(nblk > 0)
            def _go():
                pltpu.async_copy(
                    msg_hbm.at[eid_v.at[pl.ds(0, RB)]], rows_a, sem_a)

                def _gbody(b, _):
                    beven = (b % 2) == 0

                    @pl.when(b + 1 < nblk)
                    def _fire():
                        boff = (b + 1) * RB
                        @pl.when(beven)
                        def _fb():
                            pltpu.async_copy(
                                msg_hbm.at[eid_v.at[pl.ds(boff, RB)]],
                                rows_b, sem_b)
                        @pl.when(jnp.logical_not(beven))
                        def _fa():
                            pltpu.async_copy(
                                msg_hbm.at[eid_v.at[pl.ds(boff, RB)]],
                                rows_a, sem_a)

                    def _accum(rows_v, sem):
                        pltpu.make_async_copy(
                            msg_hbm.at[eid_v.at[pl.ds(0, RB)]], rows_v,
                            sem).wait()

                        def _abody(k, _):
                            base = b * RB + k * L
                            rvalid = (base + lanes) < cnt_vec
                            q = jnp.where(rvalid, loc_v[pl.ds(base, L)], NPW)
                            rowi = k * L + lanes

                            def _phase(p, col):
                                vals = plsc.load_gather(rows_v, [rowi, col])
                                plsc.addupdate_scatter(acc_v, [q, col], vals)
                                return (col + 1) & (H - 1)

                            lax.fori_loop(0, H, _phase, lanes, unroll=16)
                            return 0

                        nk = jnp.minimum(cnt - b * RB, RB)
                        lax.fori_loop(0, (nk + (L - 1)) // L, _abody, 0)

                    @pl.when(beven)
                    def _da():
                        _accum(rows_a, sem_a)

                    @pl.when(jnp.logical_not(beven))
                    def _db():
                        _accum(rows_b, sem_b)

                    return 0

                lax.fori_loop(0, nblk, _gbody, 0)

        @pl.when(even)
        def _ea():
            _with_buf(rix_a)

        @pl.when(jnp.logical_not(even))
        def _eb():
            _with_buf(rix_b)

        @pl.when(ci + 1 < NCHUNK)
        def _wait_pref():
            @pl.when(even)
            def _wb():
                pltpu.make_async_copy(
                    ridx_hbm.at[pl.ds(0, RC)], rix_b, sem_r).wait()
            @pl.when(jnp.logical_not(even))
            def _wa():
                pltpu.make_async_copy(
                    ridx_hbm.at[pl.ds(0, RC)], rix_a, sem_r).wait()

    pltpu.sync_copy(acc_v.at[pl.ds(0, NPW)], agg_hbm.at[pl.ds(lo, NPW)])


NB_A = 400
NB_E = 400
EB = 1600


def _preproj_body(nf, ws, wr, ps, pr):
    x = nf[...]
    ps[...] = jnp.dot(x, ws[...], preferred_element_type=f32)
    pr[...] = jnp.dot(x, wr[...], preferred_element_type=f32)


def _ln(x, g, b):
    mu = jnp.mean(x, -1, keepdims=True)
    xc = x - mu
    var = jnp.mean(xc * xc, -1, keepdims=True)
    return xc * lax.rsqrt(var + 1e-5) * g + b


def _edge_body(gs, gr, ef, w0, w1, w2, b0, b1, b2, g, bt, msg, ne):
    efv = ef[...]
    x = gs[...] + gr[...] + jnp.dot(efv, w0[...], preferred_element_type=f32)
    x = jnp.maximum(x + b0[...], 0.0)
    x = jnp.maximum(jnp.dot(x, w1[...], preferred_element_type=f32) + b1[...], 0.0)
    x = jnp.dot(x, w2[...], preferred_element_type=f32) + b2[...]
    y = _ln(x, g[...], bt[...])
    msg[...] = y
    ne[...] = y + efv


def _node_body(nf, agg, w0a, w0b, w1, w2, b0, b1, b2, g, bt, out):
    nfv = nf[...]
    x = (jnp.dot(nfv, w0a[...], preferred_element_type=f32)
         + jnp.dot(agg[...], w0b[...], preferred_element_type=f32))
    x = jnp.maximum(x + b0[...], 0.0)
    x = jnp.maximum(jnp.dot(x, w1[...], preferred_element_type=f32) + b1[...], 0.0)
    x = jnp.dot(x, w2[...], preferred_element_type=f32) + b2[...]
    out[...] = _ln(x, g[...], bt[...]) + nfv


def _full(shape):
    return pl.BlockSpec(shape, lambda i: (0, 0))


def kernel(senders, receivers, node_features, edge_features,
           eW0, eb0, eW1, eb1, eW2, eb2, eg, ebt,
           nW0, nb0, nW1, nb1, nW2, nb2, ng, nbt):
    w_es, w_er, w_ee = eW0[0:H], eW0[H:2 * H], eW0[2 * H:3 * H]
    nW0a, nW0b = nW0[0:H], nW0[H:2 * H]
    eb0r, eb1r, eb2r = eb0.reshape(1, H), eb1.reshape(1, H), eb2.reshape(1, H)
    egr, ebtr = eg.reshape(1, H), ebt.reshape(1, H)
    nb0r, nb1r, nb2r = nb0.reshape(1, H), nb1.reshape(1, H), nb2.reshape(1, H)
    ngr, nbtr = ng.reshape(1, H), nbt.reshape(1, H)

    ps, pr = pl.pallas_call(
        _preproj_body,
        grid=(N // NB_A,),
        in_specs=[pl.BlockSpec((NB_A, H), lambda i: (i, 0)),
                  _full((H, H)), _full((H, H))],
        out_specs=[pl.BlockSpec((NB_A, H), lambda i: (i, 0))] * 2,
        out_shape=[jax.ShapeDtypeStruct((N, H), f32)] * 2,
    )(node_features, w_es, w_er)

    gs, gr = _gather2(ps, pr, senders, receivers)

    msg, new_edge = pl.pallas_call(
        _edge_body,
        grid=(E // EB,),
        in_specs=[pl.BlockSpec((EB, H), lambda i: (i, 0))] * 3
        + [_full((H, H))] * 3 + [_full((1, H))] * 5,
        out_specs=[pl.BlockSpec((EB, H), lambda i: (i, 0))] * 2,
        out_shape=[jax.ShapeDtypeStruct((E, H), f32)] * 2,
    )(gs, gr, edge_features, w_ee, eW1, eW2, eb0r, eb1r, eb2r, egr, ebtr)

    aggp = _segsum(msg, receivers)

    new_node = pl.pallas_call(
        _node_body,
        grid=(N // NB_E,),
        in_specs=[pl.BlockSpec((NB_E, H), lambda i: (i, 0))] * 2
        + [_full((H, H))] * 4 + [_full((1, H))] * 5,
        out_specs=pl.BlockSpec((NB_E, H), lambda i: (i, 0)),
        out_shape=jax.ShapeDtypeStruct((N, H), f32),
    )(node_features, aggp, nW0a, nW0b, nW1, nW2, nb0r, nb1r, nb2r, ngr, nbtr)

    return (new_node, new_edge)
